# baseline (speedup 1.0000x reference)
import functools

import jax
import jax.numpy as jnp
from jax import lax
from jax.experimental import pallas as pl
from jax.experimental.pallas import tpu as pltpu
from jax.experimental.pallas import tpu_sc as plsc

D_, H_, W_ = 10, 200, 176
NVOX = D_ * H_ * W_
FINAL = 128
NEG = -1e30


Kt = 256
Tp = 40
T_REAL = 35


NPADS = 16384.0 * (Tp - T_REAL)


def _pad_t(xb):
    kt = xb.shape[0]
    return jnp.concatenate(
        [xb, jnp.zeros((kt, Tp - T_REAL, xb.shape[2]), xb.dtype)], axis=1)


def _layer1(xb, W1, b1):
    kt = xb.shape[0]
    x2 = xb.reshape(kt * Tp, 7)
    h = jnp.maximum(jnp.dot(x2, W1, preferred_element_type=jnp.float32) + b1, 0.0)
    return h


def _sums(h):
    s = jnp.sum(h, axis=0, keepdims=True)
    sq = jnp.sum(h * h, axis=0, keepdims=True)
    return jnp.concatenate([s, sq], axis=0)


def _stats_pass1(x_ref, W1_ref, b1_ref, out_ref):
    i = pl.program_id(0)
    xb = _pad_t(x_ref[...])
    h = _layer1(xb, W1_ref[...], b1_ref[...])
    part = _sums(h)

    @pl.when(i == 0)
    def _():
        out_ref[...] = jnp.zeros_like(out_ref)

    out_ref[...] += part


def _bn_coeffs(sums, g, bt, b, cnt):
    pc = jnp.maximum(b, 0.0)
    mean = (sums[0:1, :] - NPADS * pc) / cnt
    var = (sums[1:2, :] - NPADS * pc * pc) / cnt - mean * mean
    s = g * lax.rsqrt(var + 1e-5)
    t = bt - mean * s
    return s, t


def _negpad(row):
    t = lax.broadcasted_iota(jnp.int32, (1, Tp, 1), 1)
    return jnp.where(t < T_REAL, row.reshape(1, 1, -1), NEG)


def _vfe_split(h, kt, sums, g, bt, b, cnt):
    c = h.shape[-1]
    s, t = _bn_coeffs(sums, g, bt, b, cnt)
    hs = h * s
    hn = hs + t
    mp = jnp.max(hs.reshape(kt, Tp, c) + _negpad(t), axis=1)
    return hn, mp


def _pmask(xb):
    return (jnp.max(xb, axis=2, keepdims=True) != 0).astype(jnp.float32)


def _to_h2(xb, pm3, W1_ref, b1_ref, g1_ref, bt1_ref, s1_ref, W2_ref,
           b2_ref, cnt):
    kt = xb.shape[0]
    h1 = _layer1(xb, W1_ref[...], b1_ref[...])
    hn1, mp1 = _vfe_split(h1, kt, s1_ref[...], g1_ref[...], bt1_ref[...],
                          b1_ref[...], cnt)
    W2 = W2_ref[...].astype(jnp.bfloat16)
    a = jnp.dot(hn1.astype(jnp.bfloat16), W2[0:16],
                preferred_element_type=jnp.float32)
    bmp = jnp.dot(mp1.astype(jnp.bfloat16), W2[16:32],
                  preferred_element_type=jnp.float32)
    pre = (a.reshape(kt, Tp, 64) + bmp[:, None, :]) * pm3
    h2 = jnp.maximum(pre + b2_ref[...].reshape(1, 1, 64), 0.0)
    return h2.reshape(kt * Tp, 64)


def _stats_pass2(x_ref, W1_ref, b1_ref, g1_ref, bt1_ref, s1_ref,
                 W2_ref, b2_ref, out_ref, *, cnt):
    i = pl.program_id(0)
    xb = _pad_t(x_ref[...])
    h2 = _to_h2(xb, _pmask(xb), W1_ref, b1_ref, g1_ref, bt1_ref, s1_ref,
                W2_ref, b2_ref, cnt)
    part = _sums(h2)

    @pl.when(i == 0)
    def _():
        out_ref[...] = jnp.zeros_like(out_ref)

    out_ref[...] += part


def _final_pass(x_ref, W1_ref, b1_ref, g1_ref, bt1_ref, s1_ref,
                W2_ref, b2_ref, g2_ref, bt2_ref, s2_ref,
                Wf_ref, bf_ref, o_ref, *, cnt):
    xb = _pad_t(x_ref[...])
    kt = xb.shape[0]
    pm3 = _pmask(xb)
    h2 = _to_h2(xb, pm3, W1_ref, b1_ref, g1_ref, bt1_ref, s1_ref,
                W2_ref, b2_ref, cnt)
    hn2, mp2 = _vfe_split(h2, kt, s2_ref[...], g2_ref[...], bt2_ref[...],
                          b2_ref[...], cnt)
    Wf = Wf_ref[...].astype(jnp.bfloat16)
    a2 = jnp.dot(hn2.astype(jnp.bfloat16), Wf[0:64],
                 preferred_element_type=jnp.float32)
    bmp2 = jnp.dot(mp2.astype(jnp.bfloat16), Wf[64:128],
                   preferred_element_type=jnp.float32)
    ym = (a2.reshape(kt, Tp, FINAL) + bmp2[:, None, :]) * pm3 \
        + _negpad(bf_ref[...])
    o_ref[...] = jnp.max(ym, axis=1).T


def _vid_kernel(c_ref, out_ref):
    c = c_ref[...]
    out_ref[...] = c[:, 0:1] * (H_ * W_) + c[:, 1:2] * W_ + c[:, 2:3]


NW = 32
VPW = NVOX // NW
VPW_PAD = 11008
NPTS = 16384
NGRP = NPTS // 16
SENT = NPTS
FPW = FINAL // NW
ROWP = 16400
VC = 8000
NVC = NVOX // VC


def _sc_build_idx(vid_hbm, idx_hbm, vid_v, table_v, keybuf_v, sem):
    wid = lax.axis_index("s") * 2 + lax.axis_index("c")
    base = wid * VPW
    pltpu.sync_copy(vid_hbm, vid_v)

    def init_body(j, _):
        table_v[pl.ds(j * 16, 16)] = jnp.full((16,), SENT, jnp.int32)
        return 0

    lax.fori_loop(0, VPW_PAD // 16, init_body, 0)
    keybuf_v[pl.ds(16, 16)] = jnp.full((16,), -1, jnp.int32)

    lane = lax.iota(jnp.int32, 16)

    def body(g, _):
        pi = g * 16 + lane
        vid = vid_v[pl.ds(g * 16, 16)]
        key = vid * 16 + lane
        sk, sv = plsc.sort_key_val(key, pi)
        keybuf_v[pl.ds(0, 16)] = sk
        nxt = keybuf_v[pl.ds(1, 16)]
        svid = lax.shift_right_arithmetic(sk, 4)
        nvid = lax.shift_right_arithmetic(nxt, 4)
        loc = svid - base
        m = (svid != nvid) & (loc >= 0) & (loc < VPW)
        locc = jnp.where(m, loc, 0)
        plsc.store_scatter(table_v, [locc], sv, mask=m)
        return 0

    lax.fori_loop(0, NGRP, body, 0)
    pltpu.sync_copy(table_v.at[pl.ds(0, VPW)], idx_hbm.at[pl.ds(base, VPW)])


def _sc_materialize(oT_hbm, idx_hbm, out_hbm,
                    orow_v, idxc_v, outc_v, outc1_v, sem):
    wid = lax.axis_index("s") * 2 + lax.axis_index("c")
    fb = wid * FPW
    for f in range(FPW):
        pltpu.sync_copy(oT_hbm.at[pl.ds((fb + f) * NPTS, NPTS)],
                        orow_v.at[pl.ds(f * ROWP, NPTS)])
        orow_v[pl.ds(f * ROWP + NPTS, 16)] = jnp.zeros((16,), jnp.float32)

    outc = [outc_v, outc1_v]
    gpr = W_ // 16

    def dloop(d, _):
        for h0, hn in ((0, 48), (48, 48), (96, 48), (144, H_ - 144)):
            voff = d * (H_ * W_) + h0 * W_
            pltpu.sync_copy(idx_hbm.at[pl.ds(voff, hn * W_)],
                            idxc_v.at[pl.ds(0, hn * W_)])
            ws = []
            for f in range(FPW):
                ob = outc[f % 2]
                if f >= 2:
                    ws[f - 2].wait()

                @plsc.parallel_loop(0, hn, 1, unroll=4)
                def rowloop(h):
                    for cc in range(gpr):
                        iv = idxc_v[pl.ds((h * gpr + cc) * 16, 16)]
                        vals = plsc.load_gather(orow_v, [iv + f * ROWP])
                        ob[h, pl.ds(cc * 16, 16)] = vals

                ws.append(pltpu.async_copy(
                    ob.at[pl.ds(0, hn)],
                    out_hbm.at[0, fb + f, d, pl.ds(h0, hn), :], sem))
            ws[-2].wait()
            ws[-1].wait()
        return 0

    lax.fori_loop(0, D_, dloop, 0)


def kernel(input, voxel_coor_buffer, W1, b1, g1, bt1, W2, b2, g2, bt2, Wf, bf):
    B, K, T, C = input.shape
    N = B * K
    cnt = float(N * T)

    xp = input.reshape(N, T, C)
    coor = voxel_coor_buffer.reshape(N, 3).astype(jnp.int32)

    b1r = b1.reshape(1, 16)
    g1r = g1.reshape(1, 16)
    bt1r = bt1.reshape(1, 16)
    b2r = b2.reshape(1, 64)
    g2r = g2.reshape(1, 64)
    bt2r = bt2.reshape(1, 64)
    bfr = bf.reshape(1, FINAL)

    grid = N // Kt
    full = lambda shp: pl.BlockSpec(shp, lambda i: (0,) * len(shp))
    xspec = pl.BlockSpec((Kt, T_REAL, C), lambda i: (i, 0, 0))

    vid = pl.pallas_call(
        _vid_kernel,
        in_specs=[pl.BlockSpec((N, 3), lambda: (0, 0))],
        out_specs=pl.BlockSpec((N, 1), lambda: (0, 0)),
        out_shape=jax.ShapeDtypeStruct((N, 1), jnp.int32),
    )(coor).reshape(N)

    mesh = plsc.VectorSubcoreMesh(core_axis_name="c", subcore_axis_name="s")

    idx = pl.kernel(
        _sc_build_idx,
        mesh=mesh,
        compiler_params=pltpu.CompilerParams(needs_layout_passes=False),
        out_type=jax.ShapeDtypeStruct((NVOX,), jnp.int32),
        scratch_types=[
            pltpu.VMEM((N,), jnp.int32),
            pltpu.VMEM((VPW_PAD,), jnp.int32),
            pltpu.VMEM((32,), jnp.int32),
            pltpu.SemaphoreType.DMA,
        ],
    )(vid)

    sums1 = pl.pallas_call(
        _stats_pass1,
        grid=(grid,),
        in_specs=[xspec, full((7, 16)), full((1, 16))],
        out_specs=full((2, 16)),
        out_shape=jax.ShapeDtypeStruct((2, 16), jnp.float32),
    )(xp, W1, b1r)

    sums2 = pl.pallas_call(
        functools.partial(_stats_pass2, cnt=cnt),
        grid=(grid,),
        in_specs=[xspec, full((7, 16)), full((1, 16)), full((1, 16)),
                  full((1, 16)), full((2, 16)), full((32, 64)), full((1, 64))],
        out_specs=full((2, 64)),
        out_shape=jax.ShapeDtypeStruct((2, 64), jnp.float32),
    )(xp, W1, b1r, g1r, bt1r, sums1, W2, b2r)

    o = pl.pallas_call(
        functools.partial(_final_pass, cnt=cnt),
        grid=(grid,),
        in_specs=[xspec, full((7, 16)), full((1, 16)), full((1, 16)),
                  full((1, 16)), full((2, 16)), full((32, 64)), full((1, 64)),
                  full((1, 64)), full((1, 64)), full((2, 64)),
                  full((128, FINAL)), full((1, FINAL))],
        out_specs=pl.BlockSpec((FINAL, Kt), lambda i: (0, i)),
        out_shape=jax.ShapeDtypeStruct((FINAL, N), jnp.float32),
    )(xp, W1, b1r, g1r, bt1r, sums1, W2, b2r, g2r, bt2r, sums2, Wf, bfr)

    out5 = pl.kernel(
        _sc_materialize,
        mesh=mesh,
        compiler_params=pltpu.CompilerParams(needs_layout_passes=False),
        out_type=jax.ShapeDtypeStruct((1, FINAL, D_, H_, W_), jnp.float32),
        scratch_types=[
            pltpu.VMEM((FPW * ROWP,), jnp.float32),
            pltpu.VMEM((56 * W_,), jnp.int32),
            pltpu.VMEM((56, W_), jnp.float32),
            pltpu.VMEM((56, W_), jnp.float32),
            pltpu.SemaphoreType.DMA,
        ],
    )(o.reshape(FINAL * N), idx)

    return out5


# speedup vs baseline: 1.0204x; 1.0204x over previous
import functools

import jax
import jax.numpy as jnp
from jax import lax
from jax.experimental import pallas as pl
from jax.experimental.pallas import tpu as pltpu
from jax.experimental.pallas import tpu_sc as plsc

D_, H_, W_ = 10, 200, 176
NVOX = D_ * H_ * W_
FINAL = 128
NEG = -1e30


Kt = 256
Tp = 40
T_REAL = 35


NPADS = 16384.0 * (Tp - T_REAL)


def _pad_t(xb):
    kt = xb.shape[0]
    return jnp.concatenate(
        [xb, jnp.zeros((kt, Tp - T_REAL, xb.shape[2]), xb.dtype)], axis=1)


def _layer1(xb, W1, b1):
    kt = xb.shape[0]
    x2 = xb.reshape(kt * Tp, 7)
    h = jnp.maximum(jnp.dot(x2, W1, preferred_element_type=jnp.float32) + b1, 0.0)
    return h


def _sums(h):
    s = jnp.sum(h, axis=0, keepdims=True)
    sq = jnp.sum(h * h, axis=0, keepdims=True)
    return jnp.concatenate([s, sq], axis=0)


def _stats_pass1(x_ref, W1_ref, b1_ref, out_ref):
    i = pl.program_id(0)
    xb = _pad_t(x_ref[...])
    h = _layer1(xb, W1_ref[...], b1_ref[...])
    part = _sums(h)

    @pl.when(i == 0)
    def _():
        out_ref[...] = jnp.zeros_like(out_ref)

    out_ref[...] += part


def _bn_coeffs(sums, g, bt, b, cnt):
    pc = jnp.maximum(b, 0.0)
    mean = (sums[0:1, :] - NPADS * pc) / cnt
    var = (sums[1:2, :] - NPADS * pc * pc) / cnt - mean * mean
    s = g * lax.rsqrt(var + 1e-5)
    t = bt - mean * s
    return s, t


def _negpad(row):
    t = lax.broadcasted_iota(jnp.int32, (1, Tp, 1), 1)
    return jnp.where(t < T_REAL, row.reshape(1, 1, -1), NEG)


def _vfe_split(h, kt, sums, g, bt, b, cnt):
    c = h.shape[-1]
    s, t = _bn_coeffs(sums, g, bt, b, cnt)
    hs = h * s
    hn = hs + t
    mp = jnp.max(hs.reshape(kt, Tp, c) + _negpad(t), axis=1)
    return hn, mp


def _pmask(xb):
    return (jnp.max(xb, axis=2, keepdims=True) != 0).astype(jnp.float32)


def _to_h2(xb, pm3, W1_ref, b1_ref, g1_ref, bt1_ref, s1_ref, W2_ref,
           b2_ref, cnt):
    kt = xb.shape[0]
    h1 = _layer1(xb, W1_ref[...], b1_ref[...])
    hn1, mp1 = _vfe_split(h1, kt, s1_ref[...], g1_ref[...], bt1_ref[...],
                          b1_ref[...], cnt)
    W2 = W2_ref[...]
    a = jnp.dot(hn1, W2[0:16], preferred_element_type=jnp.float32)
    bmp = jnp.dot(mp1, W2[16:32], preferred_element_type=jnp.float32)
    pre = (a.reshape(kt, Tp, 64) + bmp[:, None, :]) * pm3
    h2 = jnp.maximum(pre + b2_ref[...].reshape(1, 1, 64), 0.0)
    return h2.reshape(kt * Tp, 64)


def _stats_pass2(x_ref, W1_ref, b1_ref, g1_ref, bt1_ref, s1_ref,
                 W2_ref, b2_ref, out_ref, *, cnt):
    i = pl.program_id(0)
    xb = _pad_t(x_ref[...])
    h2 = _to_h2(xb, _pmask(xb), W1_ref, b1_ref, g1_ref, bt1_ref, s1_ref,
                W2_ref, b2_ref, cnt)
    part = _sums(h2)

    @pl.when(i == 0)
    def _():
        out_ref[...] = jnp.zeros_like(out_ref)

    out_ref[...] += part


def _final_pass(x_ref, W1_ref, b1_ref, g1_ref, bt1_ref, s1_ref,
                W2_ref, b2_ref, g2_ref, bt2_ref, s2_ref,
                Wf_ref, bf_ref, o_ref, *, cnt):
    xb = _pad_t(x_ref[...])
    kt = xb.shape[0]
    pm3 = _pmask(xb)
    h2 = _to_h2(xb, pm3, W1_ref, b1_ref, g1_ref, bt1_ref, s1_ref,
                W2_ref, b2_ref, cnt)
    hn2, mp2 = _vfe_split(h2, kt, s2_ref[...], g2_ref[...], bt2_ref[...],
                          b2_ref[...], cnt)
    Wf = Wf_ref[...]
    a2 = jnp.dot(hn2, Wf[0:64], preferred_element_type=jnp.float32)
    bmp2 = jnp.dot(mp2, Wf[64:128], preferred_element_type=jnp.float32)
    ym = (a2.reshape(kt, Tp, FINAL) + bmp2[:, None, :]) * pm3 \
        + _negpad(bf_ref[...])
    o_ref[...] = jnp.max(ym, axis=1).T


def _vid_kernel(c_ref, out_ref):
    c = c_ref[...]
    out_ref[...] = c[:, 0:1] * (H_ * W_) + c[:, 1:2] * W_ + c[:, 2:3]


NW = 32
VPW = NVOX // NW
VPW_PAD = 11008
NPTS = 16384
NGRP = NPTS // 16
SENT = NPTS
FPW = FINAL // NW
ROWP = 16400
VC = 8000
NVC = NVOX // VC


def _sc_build_idx(vid_hbm, idx_hbm, vid_v, table_v, keybuf_v, sem):
    wid = lax.axis_index("s") * 2 + lax.axis_index("c")
    base = wid * VPW
    pltpu.sync_copy(vid_hbm, vid_v)

    def init_body(j, _):
        table_v[pl.ds(j * 16, 16)] = jnp.full((16,), SENT, jnp.int32)
        return 0

    lax.fori_loop(0, VPW_PAD // 16, init_body, 0)
    keybuf_v[pl.ds(16, 16)] = jnp.full((16,), -1, jnp.int32)

    lane = lax.iota(jnp.int32, 16)

    def body(g, _):
        pi = g * 16 + lane
        vid = vid_v[pl.ds(g * 16, 16)]
        key = vid * 16 + lane
        sk, sv = plsc.sort_key_val(key, pi)
        keybuf_v[pl.ds(0, 16)] = sk
        nxt = keybuf_v[pl.ds(1, 16)]
        svid = lax.shift_right_arithmetic(sk, 4)
        nvid = lax.shift_right_arithmetic(nxt, 4)
        loc = svid - base
        m = (svid != nvid) & (loc >= 0) & (loc < VPW)
        locc = jnp.where(m, loc, 0)
        plsc.store_scatter(table_v, [locc], sv, mask=m)
        return 0

    lax.fori_loop(0, NGRP, body, 0)
    pltpu.sync_copy(table_v.at[pl.ds(0, VPW)], idx_hbm.at[pl.ds(base, VPW)])


def _sc_materialize(oT_hbm, idx_hbm, out_hbm,
                    orow_v, idxc_v, outc_v, outc1_v, sem):
    wid = lax.axis_index("s") * 2 + lax.axis_index("c")
    fb = wid * FPW
    for f in range(FPW):
        pltpu.sync_copy(oT_hbm.at[pl.ds((fb + f) * NPTS, NPTS)],
                        orow_v.at[pl.ds(f * ROWP, NPTS)])
        orow_v[pl.ds(f * ROWP + NPTS, 16)] = jnp.zeros((16,), jnp.float32)

    outc = [outc_v, outc1_v]
    gpr = W_ // 16

    def dloop(d, _):
        for h0, hn in ((0, 48), (48, 48), (96, 48), (144, H_ - 144)):
            voff = d * (H_ * W_) + h0 * W_
            pltpu.sync_copy(idx_hbm.at[pl.ds(voff, hn * W_)],
                            idxc_v.at[pl.ds(0, hn * W_)])
            ws = []
            for f in range(FPW):
                ob = outc[f % 2]
                if f >= 2:
                    ws[f - 2].wait()

                @plsc.parallel_loop(0, hn, 1, unroll=4)
                def rowloop(h):
                    for cc in range(gpr):
                        iv = idxc_v[pl.ds((h * gpr + cc) * 16, 16)]
                        vals = plsc.load_gather(orow_v, [iv + f * ROWP])
                        ob[h, pl.ds(cc * 16, 16)] = vals

                ws.append(pltpu.async_copy(
                    ob.at[pl.ds(0, hn)],
                    out_hbm.at[0, fb + f, d, pl.ds(h0, hn), :], sem))
            ws[-2].wait()
            ws[-1].wait()
        return 0

    lax.fori_loop(0, D_, dloop, 0)


def kernel(input, voxel_coor_buffer, W1, b1, g1, bt1, W2, b2, g2, bt2, Wf, bf):
    B, K, T, C = input.shape
    N = B * K
    cnt = float(N * T)

    xp = input.reshape(N, T, C)
    coor = voxel_coor_buffer.reshape(N, 3).astype(jnp.int32)

    b1r = b1.reshape(1, 16)
    g1r = g1.reshape(1, 16)
    bt1r = bt1.reshape(1, 16)
    b2r = b2.reshape(1, 64)
    g2r = g2.reshape(1, 64)
    bt2r = bt2.reshape(1, 64)
    bfr = bf.reshape(1, FINAL)

    grid = N // Kt
    full = lambda shp: pl.BlockSpec(shp, lambda i: (0,) * len(shp))
    xspec = pl.BlockSpec((Kt, T_REAL, C), lambda i: (i, 0, 0))

    vid = pl.pallas_call(
        _vid_kernel,
        in_specs=[pl.BlockSpec((N, 3), lambda: (0, 0))],
        out_specs=pl.BlockSpec((N, 1), lambda: (0, 0)),
        out_shape=jax.ShapeDtypeStruct((N, 1), jnp.int32),
    )(coor).reshape(N)

    mesh = plsc.VectorSubcoreMesh(core_axis_name="c", subcore_axis_name="s")

    idx = pl.kernel(
        _sc_build_idx,
        mesh=mesh,
        compiler_params=pltpu.CompilerParams(needs_layout_passes=False),
        out_type=jax.ShapeDtypeStruct((NVOX,), jnp.int32),
        scratch_types=[
            pltpu.VMEM((N,), jnp.int32),
            pltpu.VMEM((VPW_PAD,), jnp.int32),
            pltpu.VMEM((32,), jnp.int32),
            pltpu.SemaphoreType.DMA,
        ],
    )(vid)

    sums1 = pl.pallas_call(
        _stats_pass1,
        grid=(grid,),
        in_specs=[xspec, full((7, 16)), full((1, 16))],
        out_specs=full((2, 16)),
        out_shape=jax.ShapeDtypeStruct((2, 16), jnp.float32),
    )(xp, W1, b1r)

    sums2 = pl.pallas_call(
        functools.partial(_stats_pass2, cnt=cnt),
        grid=(grid,),
        in_specs=[xspec, full((7, 16)), full((1, 16)), full((1, 16)),
                  full((1, 16)), full((2, 16)), full((32, 64)), full((1, 64))],
        out_specs=full((2, 64)),
        out_shape=jax.ShapeDtypeStruct((2, 64), jnp.float32),
    )(xp, W1, b1r, g1r, bt1r, sums1, W2, b2r)

    o = pl.pallas_call(
        functools.partial(_final_pass, cnt=cnt),
        grid=(grid,),
        in_specs=[xspec, full((7, 16)), full((1, 16)), full((1, 16)),
                  full((1, 16)), full((2, 16)), full((32, 64)), full((1, 64)),
                  full((1, 64)), full((1, 64)), full((2, 64)),
                  full((128, FINAL)), full((1, FINAL))],
        out_specs=pl.BlockSpec((FINAL, Kt), lambda i: (0, i)),
        out_shape=jax.ShapeDtypeStruct((FINAL, N), jnp.float32),
    )(xp, W1, b1r, g1r, bt1r, sums1, W2, b2r, g2r, bt2r, sums2, Wf, bfr)

    out5 = pl.kernel(
        _sc_materialize,
        mesh=mesh,
        compiler_params=pltpu.CompilerParams(needs_layout_passes=False),
        out_type=jax.ShapeDtypeStruct((1, FINAL, D_, H_, W_), jnp.float32),
        scratch_types=[
            pltpu.VMEM((FPW * ROWP,), jnp.float32),
            pltpu.VMEM((56 * W_,), jnp.int32),
            pltpu.VMEM((56, W_), jnp.float32),
            pltpu.VMEM((56, W_), jnp.float32),
            pltpu.SemaphoreType.DMA,
        ],
    )(o.reshape(FINAL * N), idx)

    return out5


# speedup vs baseline: 1.0298x; 1.0092x over previous
import functools

import jax
import jax.numpy as jnp
from jax import lax
from jax.experimental import pallas as pl
from jax.experimental.pallas import tpu as pltpu
from jax.experimental.pallas import tpu_sc as plsc

D_, H_, W_ = 10, 200, 176
NVOX = D_ * H_ * W_
FINAL = 128
NEG = -1e30


Kt = 256
Tp = 40
T_REAL = 35


NPADS = 16384.0 * (Tp - T_REAL)


def _pad_t(xb):
    kt = xb.shape[0]
    return jnp.concatenate(
        [xb, jnp.zeros((kt, Tp - T_REAL, xb.shape[2]), xb.dtype)], axis=1)


def _layer1(xb, W1, b1):
    kt = xb.shape[0]
    x2 = xb.reshape(kt * Tp, 7)
    h = jnp.maximum(jnp.dot(x2, W1, preferred_element_type=jnp.float32) + b1, 0.0)
    return h


def _sums(h):
    s = jnp.sum(h, axis=0, keepdims=True)
    sq = jnp.sum(h * h, axis=0, keepdims=True)
    return jnp.concatenate([s, sq], axis=0)


def _stats_pass1(x_ref, W1_ref, b1_ref, out_ref):
    i = pl.program_id(0)
    xb = _pad_t(x_ref[...])
    h = _layer1(xb, W1_ref[...], b1_ref[...])
    part = _sums(h)

    @pl.when(i == 0)
    def _():
        out_ref[...] = jnp.zeros_like(out_ref)

    out_ref[...] += part


def _bn_coeffs(sums, g, bt, b, cnt):
    pc = jnp.maximum(b, 0.0)
    mean = (sums[0:1, :] - NPADS * pc) / cnt
    var = (sums[1:2, :] - NPADS * pc * pc) / cnt - mean * mean
    s = g * lax.rsqrt(var + 1e-5)
    t = bt - mean * s
    return s, t


def _negpad(row):
    t = lax.broadcasted_iota(jnp.int32, (1, Tp, 1), 1)
    return jnp.where(t < T_REAL, row.reshape(1, 1, -1), NEG)


def _vfe_split(h, kt, sums, g, bt, b, cnt):
    c = h.shape[-1]
    s, t = _bn_coeffs(sums, g, bt, b, cnt)
    hs = h * s
    hn = hs + t
    mp = jnp.max(hs.reshape(kt, Tp, c) + _negpad(t), axis=1)
    return hn, mp


def _pmask(xb):
    return (jnp.max(xb, axis=2, keepdims=True) != 0).astype(jnp.float32)


def _to_h2(xb, pm3, W1_ref, b1_ref, g1_ref, bt1_ref, s1_ref, W2_ref,
           b2_ref, cnt):
    kt = xb.shape[0]
    h1 = _layer1(xb, W1_ref[...], b1_ref[...])
    hn1, mp1 = _vfe_split(h1, kt, s1_ref[...], g1_ref[...], bt1_ref[...],
                          b1_ref[...], cnt)
    W2 = W2_ref[...]
    a = jnp.dot(hn1, W2[0:16], preferred_element_type=jnp.float32)
    bmp = jnp.dot(mp1, W2[16:32], preferred_element_type=jnp.float32)
    pre = (a.reshape(kt, Tp, 64) + bmp[:, None, :]) * pm3
    h2 = jnp.maximum(pre + b2_ref[...].reshape(1, 1, 64), 0.0)
    return h2.reshape(kt * Tp, 64)


def _stats_pass2(x_ref, W1_ref, b1_ref, g1_ref, bt1_ref, s1_ref,
                 W2_ref, b2_ref, out_ref, *, cnt):
    i = pl.program_id(0)
    xb = _pad_t(x_ref[...])
    h2 = _to_h2(xb, _pmask(xb), W1_ref, b1_ref, g1_ref, bt1_ref, s1_ref,
                W2_ref, b2_ref, cnt)
    part = _sums(h2)

    @pl.when(i == 0)
    def _():
        out_ref[...] = jnp.zeros_like(out_ref)

    out_ref[...] += part


def _final_pass(x_ref, W1_ref, b1_ref, g1_ref, bt1_ref, s1_ref,
                W2_ref, b2_ref, g2_ref, bt2_ref, s2_ref,
                Wf_ref, bf_ref, o_ref, *, cnt):
    xb = _pad_t(x_ref[...])
    kt = xb.shape[0]
    pm3 = _pmask(xb)
    h2 = _to_h2(xb, pm3, W1_ref, b1_ref, g1_ref, bt1_ref, s1_ref,
                W2_ref, b2_ref, cnt)
    hn2, mp2 = _vfe_split(h2, kt, s2_ref[...], g2_ref[...], bt2_ref[...],
                          b2_ref[...], cnt)
    Wf = Wf_ref[...]
    a2 = jnp.dot(hn2, Wf[0:64], preferred_element_type=jnp.float32)
    bmp2 = jnp.dot(mp2, Wf[64:128], preferred_element_type=jnp.float32)
    ym = (a2.reshape(kt, Tp, FINAL) + bmp2[:, None, :]) * pm3 \
        + _negpad(bf_ref[...])
    o_ref[...] = jnp.max(ym, axis=1).T


def _vid_kernel(c_ref, out_ref):
    c = c_ref[...]
    out_ref[...] = c[:, 0:1] * (H_ * W_) + c[:, 1:2] * W_ + c[:, 2:3]


NW = 32
VPW = NVOX // NW
VPW_PAD = 11008
NPTS = 16384
NGRP = NPTS // 16
SENT = NPTS
FPW = FINAL // NW
ROWP = 16400
VC = 8000
NVC = NVOX // VC


def _sc_build_idx(vid_hbm, idx_hbm, vid_v, table_v, keybuf_v, sem):
    wid = lax.axis_index("s") * 2 + lax.axis_index("c")
    base = wid * VPW
    pltpu.sync_copy(vid_hbm, vid_v)

    def init_body(j, _):
        table_v[pl.ds(j * 16, 16)] = jnp.full((16,), SENT, jnp.int32)
        return 0

    lax.fori_loop(0, VPW_PAD // 16, init_body, 0)
    keybuf_v[pl.ds(16, 16)] = jnp.full((16,), -1, jnp.int32)

    lane = lax.iota(jnp.int32, 16)

    def body(g, _):
        pi = g * 16 + lane
        vid = vid_v[pl.ds(g * 16, 16)]
        key = vid * 16 + lane
        sk, sv = plsc.sort_key_val(key, pi)
        keybuf_v[pl.ds(0, 16)] = sk
        nxt = keybuf_v[pl.ds(1, 16)]
        svid = lax.shift_right_arithmetic(sk, 4)
        nvid = lax.shift_right_arithmetic(nxt, 4)
        loc = svid - base
        m = (svid != nvid) & (loc >= 0) & (loc < VPW)
        locc = jnp.where(m, loc, 0)
        plsc.store_scatter(table_v, [locc], sv, mask=m)
        return 0

    lax.fori_loop(0, NGRP, body, 0)
    pltpu.sync_copy(table_v.at[pl.ds(0, VPW)], idx_hbm.at[pl.ds(base, VPW)])


def _sc_materialize(oT_hbm, idx_hbm, out_hbm,
                    orow_v, idxc_v, outc_v, outc1_v, sem):
    wid = lax.axis_index("s") * 2 + lax.axis_index("c")
    fb = wid * FPW
    for f in range(FPW):
        pltpu.sync_copy(oT_hbm.at[pl.ds((fb + f) * NPTS, NPTS)],
                        orow_v.at[pl.ds(f * ROWP, NPTS)])
        orow_v[pl.ds(f * ROWP + NPTS, 16)] = jnp.zeros((16,), jnp.float32)

    outc = [outc_v, outc1_v]
    gpr = W_ // 16

    def dloop(d, _):
        for h0, hn in ((0, 48), (48, 48), (96, 48), (144, H_ - 144)):
            voff = d * (H_ * W_) + h0 * W_
            pltpu.sync_copy(idx_hbm.at[pl.ds(voff, hn * W_)],
                            idxc_v.at[pl.ds(0, hn * W_)])
            ws = []
            for f in range(FPW):
                ob = outc[f % 2]
                if f >= 2:
                    ws[f - 2].wait()

                @plsc.parallel_loop(0, hn, 1, unroll=8)
                def rowloop(h):
                    for cc in range(gpr):
                        iv = idxc_v[pl.ds((h * gpr + cc) * 16, 16)]
                        vals = plsc.load_gather(orow_v, [iv + f * ROWP])
                        ob[h, pl.ds(cc * 16, 16)] = vals

                ws.append(pltpu.async_copy(
                    ob.at[pl.ds(0, hn)],
                    out_hbm.at[0, fb + f, d, pl.ds(h0, hn), :], sem))
            ws[-2].wait()
            ws[-1].wait()
        return 0

    lax.fori_loop(0, D_, dloop, 0)


def kernel(input, voxel_coor_buffer, W1, b1, g1, bt1, W2, b2, g2, bt2, Wf, bf):
    B, K, T, C = input.shape
    N = B * K
    cnt = float(N * T)

    xp = input.reshape(N, T, C)
    coor = voxel_coor_buffer.reshape(N, 3).astype(jnp.int32)

    b1r = b1.reshape(1, 16)
    g1r = g1.reshape(1, 16)
    bt1r = bt1.reshape(1, 16)
    b2r = b2.reshape(1, 64)
    g2r = g2.reshape(1, 64)
    bt2r = bt2.reshape(1, 64)
    bfr = bf.reshape(1, FINAL)

    grid = N // Kt
    full = lambda shp: pl.BlockSpec(shp, lambda i: (0,) * len(shp))
    xspec = pl.BlockSpec((Kt, T_REAL, C), lambda i: (i, 0, 0))

    vid = pl.pallas_call(
        _vid_kernel,
        in_specs=[pl.BlockSpec((N, 3), lambda: (0, 0))],
        out_specs=pl.BlockSpec((N, 1), lambda: (0, 0)),
        out_shape=jax.ShapeDtypeStruct((N, 1), jnp.int32),
    )(coor).reshape(N)

    mesh = plsc.VectorSubcoreMesh(core_axis_name="c", subcore_axis_name="s")

    idx = pl.kernel(
        _sc_build_idx,
        mesh=mesh,
        compiler_params=pltpu.CompilerParams(needs_layout_passes=False),
        out_type=jax.ShapeDtypeStruct((NVOX,), jnp.int32),
        scratch_types=[
            pltpu.VMEM((N,), jnp.int32),
            pltpu.VMEM((VPW_PAD,), jnp.int32),
            pltpu.VMEM((32,), jnp.int32),
            pltpu.SemaphoreType.DMA,
        ],
    )(vid)

    sums1 = pl.pallas_call(
        _stats_pass1,
        grid=(grid,),
        in_specs=[xspec, full((7, 16)), full((1, 16))],
        out_specs=full((2, 16)),
        out_shape=jax.ShapeDtypeStruct((2, 16), jnp.float32),
    )(xp, W1, b1r)

    sums1, idx = lax.optimization_barrier((sums1, idx))

    sums2 = pl.pallas_call(
        functools.partial(_stats_pass2, cnt=cnt),
        grid=(grid,),
        in_specs=[xspec, full((7, 16)), full((1, 16)), full((1, 16)),
                  full((1, 16)), full((2, 16)), full((32, 64)), full((1, 64))],
        out_specs=full((2, 64)),
        out_shape=jax.ShapeDtypeStruct((2, 64), jnp.float32),
    )(xp, W1, b1r, g1r, bt1r, sums1, W2, b2r)

    o = pl.pallas_call(
        functools.partial(_final_pass, cnt=cnt),
        grid=(grid,),
        in_specs=[xspec, full((7, 16)), full((1, 16)), full((1, 16)),
                  full((1, 16)), full((2, 16)), full((32, 64)), full((1, 64)),
                  full((1, 64)), full((1, 64)), full((2, 64)),
                  full((128, FINAL)), full((1, FINAL))],
        out_specs=pl.BlockSpec((FINAL, Kt), lambda i: (0, i)),
        out_shape=jax.ShapeDtypeStruct((FINAL, N), jnp.float32),
    )(xp, W1, b1r, g1r, bt1r, sums1, W2, b2r, g2r, bt2r, sums2, Wf, bfr)

    out5 = pl.kernel(
        _sc_materialize,
        mesh=mesh,
        compiler_params=pltpu.CompilerParams(needs_layout_passes=False),
        out_type=jax.ShapeDtypeStruct((1, FINAL, D_, H_, W_), jnp.float32),
        scratch_types=[
            pltpu.VMEM((FPW * ROWP,), jnp.float32),
            pltpu.VMEM((56 * W_,), jnp.int32),
            pltpu.VMEM((56, W_), jnp.float32),
            pltpu.VMEM((56, W_), jnp.float32),
            pltpu.SemaphoreType.DMA,
        ],
    )(o.reshape(FINAL * N), idx)

    return out5


# speedup vs baseline: 1.0802x; 1.0490x over previous
import functools

import jax
import jax.numpy as jnp
from jax import lax
from jax.experimental import pallas as pl
from jax.experimental.pallas import tpu as pltpu
from jax.experimental.pallas import tpu_sc as plsc

D_, H_, W_ = 10, 200, 176
NVOX = D_ * H_ * W_
FINAL = 128
NEG = -1e30


Kt = 256
Tp = 40
T_REAL = 35


NPADS = 16384.0 * (Tp - T_REAL)


def _pad_t(xb):
    kt = xb.shape[0]
    return jnp.concatenate(
        [xb, jnp.zeros((kt, Tp - T_REAL, xb.shape[2]), xb.dtype)], axis=1)


def _layer1(xb, W1, b1):
    kt = xb.shape[0]
    x2 = xb.reshape(kt * Tp, 7)
    h = jnp.maximum(jnp.dot(x2, W1, preferred_element_type=jnp.float32) + b1, 0.0)
    return h


def _sums(h):
    s = jnp.sum(h, axis=0, keepdims=True)
    sq = jnp.sum(h * h, axis=0, keepdims=True)
    return jnp.concatenate([s, sq], axis=0)


def _stats_pass1(x_ref, W_ref, b_ref, out_ref):
    i = pl.program_id(0)
    h = jnp.maximum(
        jnp.dot(x_ref[...], W_ref[...], preferred_element_type=jnp.float32)
        + b_ref[...], 0.0)
    part = _sums(h)

    @pl.when(i == 0)
    def _():
        out_ref[...] = jnp.zeros_like(out_ref)

    out_ref[...] += part


def _bn_coeffs(sums, g, bt, b, cnt, pads):
    pc = jnp.maximum(b, 0.0)
    mean = (sums[0:1, :] - pads * pc) / cnt
    var = (sums[1:2, :] - pads * pc * pc) / cnt - mean * mean
    s = g * lax.rsqrt(var + 1e-5)
    t = bt - mean * s
    return s, t


def _negpad(row):
    t = lax.broadcasted_iota(jnp.int32, (1, Tp, 1), 1)
    return jnp.where(t < T_REAL, row.reshape(1, 1, -1), NEG)


def _vfe_split(h, kt, sums, g, bt, b, cnt, pads):
    c = h.shape[-1]
    s, t = _bn_coeffs(sums, g, bt, b, cnt, pads)
    hs = h * s
    hn = hs + t
    mp = jnp.max(hs.reshape(kt, Tp, c) + _negpad(t), axis=1)
    return hn, mp


def _pmask(xb):
    return (jnp.max(xb, axis=2, keepdims=True) != 0).astype(jnp.float32)


def _to_h2(xb, pm3, W1_ref, b1_ref, g1_ref, bt1_ref, s1_ref, W2_ref,
           b2_ref, cnt):
    kt = xb.shape[0]
    h1 = _layer1(xb, W1_ref[...], b1_ref[...])
    s1_16 = jnp.sum(s1_ref[...].reshape(2, T_REAL, 16), axis=1)
    hn1, mp1 = _vfe_split(h1, kt, s1_16, g1_ref[...], bt1_ref[...],
                          b1_ref[...], cnt, 0.0)
    W2 = W2_ref[...]
    a = jnp.dot(hn1, W2[0:16], preferred_element_type=jnp.float32)
    bmp = jnp.dot(mp1, W2[16:32], preferred_element_type=jnp.float32)
    pre = (a.reshape(kt, Tp, 64) + bmp[:, None, :]) * pm3
    h2 = jnp.maximum(pre + b2_ref[...].reshape(1, 1, 64), 0.0)
    return h2.reshape(kt * Tp, 64)


def _stats_pass2(x_ref, W1_ref, b1_ref, g1_ref, bt1_ref, s1_ref,
                 W2_ref, b2_ref, out_ref, *, cnt):
    i = pl.program_id(0)
    xb = _pad_t(x_ref[...])
    h2 = _to_h2(xb, _pmask(xb), W1_ref, b1_ref, g1_ref, bt1_ref, s1_ref,
                W2_ref, b2_ref, cnt)
    part = _sums(h2)

    @pl.when(i == 0)
    def _():
        out_ref[...] = jnp.zeros_like(out_ref)

    out_ref[...] += part


def _final_pass(x_ref, W1_ref, b1_ref, g1_ref, bt1_ref, s1_ref,
                W2_ref, b2_ref, g2_ref, bt2_ref, s2_ref,
                Wf_ref, bf_ref, o_ref, *, cnt):
    xb = _pad_t(x_ref[...])
    kt = xb.shape[0]
    pm3 = _pmask(xb)
    h2 = _to_h2(xb, pm3, W1_ref, b1_ref, g1_ref, bt1_ref, s1_ref,
                W2_ref, b2_ref, cnt)
    hn2, mp2 = _vfe_split(h2, kt, s2_ref[...], g2_ref[...], bt2_ref[...],
                          b2_ref[...], cnt, NPADS)
    Wf = Wf_ref[...]
    a2 = jnp.dot(hn2, Wf[0:64], preferred_element_type=jnp.float32)
    bmp2 = jnp.dot(mp2, Wf[64:128], preferred_element_type=jnp.float32)
    ym = (a2.reshape(kt, Tp, FINAL) + bmp2[:, None, :]) * pm3 \
        + _negpad(bf_ref[...])
    o_ref[...] = jnp.max(ym, axis=1).T


def _vid_kernel(c_ref, out_ref):
    c = c_ref[...]
    out_ref[...] = c[:, 0:1] * (H_ * W_) + c[:, 1:2] * W_ + c[:, 2:3]


NW = 32
VPW = NVOX // NW
VPW_PAD = 11008
NPTS = 16384
NGRP = NPTS // 16
SENT = NPTS
FPW = FINAL // NW
ROWP = 16400
VC = 8000
NVC = NVOX // VC


def _sc_build_idx(vid_hbm, idx_hbm, vid_v, table_v, keybuf_v, sem):
    wid = lax.axis_index("s") * 2 + lax.axis_index("c")
    base = wid * VPW
    pltpu.sync_copy(vid_hbm, vid_v)

    def init_body(j, _):
        table_v[pl.ds(j * 16, 16)] = jnp.full((16,), SENT, jnp.int32)
        return 0

    lax.fori_loop(0, VPW_PAD // 16, init_body, 0)
    keybuf_v[pl.ds(16, 16)] = jnp.full((16,), -1, jnp.int32)

    lane = lax.iota(jnp.int32, 16)

    def body(g, _):
        pi = g * 16 + lane
        vid = vid_v[pl.ds(g * 16, 16)]
        key = vid * 16 + lane
        sk, sv = plsc.sort_key_val(key, pi)
        keybuf_v[pl.ds(0, 16)] = sk
        nxt = keybuf_v[pl.ds(1, 16)]
        svid = lax.shift_right_arithmetic(sk, 4)
        nvid = lax.shift_right_arithmetic(nxt, 4)
        loc = svid - base
        m = (svid != nvid) & (loc >= 0) & (loc < VPW)
        locc = jnp.where(m, loc, 0)
        plsc.store_scatter(table_v, [locc], sv, mask=m)
        return 0

    lax.fori_loop(0, NGRP, body, 0)
    pltpu.sync_copy(table_v.at[pl.ds(0, VPW)], idx_hbm.at[pl.ds(base, VPW)])


def _sc_materialize(oT_hbm, idx_hbm, out_hbm,
                    orow_v, idxc_v, outc_v, outc1_v, sem):
    wid = lax.axis_index("s") * 2 + lax.axis_index("c")
    fb = wid * FPW
    for f in range(FPW):
        pltpu.sync_copy(oT_hbm.at[pl.ds((fb + f) * NPTS, NPTS)],
                        orow_v.at[pl.ds(f * ROWP, NPTS)])
        orow_v[pl.ds(f * ROWP + NPTS, 16)] = jnp.zeros((16,), jnp.float32)

    outc = [outc_v, outc1_v]
    gpr = W_ // 16

    def dloop(d, _):
        for h0, hn in ((0, 48), (48, 48), (96, 48), (144, H_ - 144)):
            voff = d * (H_ * W_) + h0 * W_
            pltpu.sync_copy(idx_hbm.at[pl.ds(voff, hn * W_)],
                            idxc_v.at[pl.ds(0, hn * W_)])
            ws = []
            for f in range(FPW):
                ob = outc[f % 2]
                if f >= 2:
                    ws[f - 2].wait()

                @plsc.parallel_loop(0, hn, 1, unroll=8)
                def rowloop(h):
                    for cc in range(gpr):
                        iv = idxc_v[pl.ds((h * gpr + cc) * 16, 16)]
                        vals = plsc.load_gather(orow_v, [iv + f * ROWP])
                        ob[h, pl.ds(cc * 16, 16)] = vals

                ws.append(pltpu.async_copy(
                    ob.at[pl.ds(0, hn)],
                    out_hbm.at[0, fb + f, d, pl.ds(h0, hn), :], sem))
            ws[-2].wait()
            ws[-1].wait()
        return 0

    lax.fori_loop(0, D_, dloop, 0)


def kernel(input, voxel_coor_buffer, W1, b1, g1, bt1, W2, b2, g2, bt2, Wf, bf):
    B, K, T, C = input.shape
    N = B * K
    cnt = float(N * T)

    xp = input.reshape(N, T, C)
    xpk = input.reshape(N, T * C)
    W1b = jnp.kron(jnp.eye(T, dtype=jnp.float32), W1)
    b1b = jnp.tile(b1.reshape(1, 16), (1, T))
    coor = voxel_coor_buffer.reshape(N, 3).astype(jnp.int32)

    b1r = b1.reshape(1, 16)
    g1r = g1.reshape(1, 16)
    bt1r = bt1.reshape(1, 16)
    b2r = b2.reshape(1, 64)
    g2r = g2.reshape(1, 64)
    bt2r = bt2.reshape(1, 64)
    bfr = bf.reshape(1, FINAL)

    grid = N // Kt
    full = lambda shp: pl.BlockSpec(shp, lambda i: (0,) * len(shp))
    xspec = pl.BlockSpec((Kt, T_REAL, C), lambda i: (i, 0, 0))

    vid = pl.pallas_call(
        _vid_kernel,
        in_specs=[pl.BlockSpec((N, 3), lambda: (0, 0))],
        out_specs=pl.BlockSpec((N, 1), lambda: (0, 0)),
        out_shape=jax.ShapeDtypeStruct((N, 1), jnp.int32),
    )(coor).reshape(N)

    mesh = plsc.VectorSubcoreMesh(core_axis_name="c", subcore_axis_name="s")

    idx = pl.kernel(
        _sc_build_idx,
        mesh=mesh,
        compiler_params=pltpu.CompilerParams(needs_layout_passes=False),
        out_type=jax.ShapeDtypeStruct((NVOX,), jnp.int32),
        scratch_types=[
            pltpu.VMEM((N,), jnp.int32),
            pltpu.VMEM((VPW_PAD,), jnp.int32),
            pltpu.VMEM((32,), jnp.int32),
            pltpu.SemaphoreType.DMA,
        ],
    )(vid)

    sums1 = pl.pallas_call(
        _stats_pass1,
        grid=(grid,),
        in_specs=[pl.BlockSpec((Kt, T * C), lambda i: (i, 0)),
                  full((T * C, T * 16)), full((1, T * 16))],
        out_specs=full((2, T * 16)),
        out_shape=jax.ShapeDtypeStruct((2, T * 16), jnp.float32),
    )(xpk, W1b, b1b)

    sums1, idx = lax.optimization_barrier((sums1, idx))

    sums2 = pl.pallas_call(
        functools.partial(_stats_pass2, cnt=cnt),
        grid=(grid,),
        in_specs=[xspec, full((7, 16)), full((1, 16)), full((1, 16)),
                  full((1, 16)), full((2, T * 16)), full((32, 64)),
                  full((1, 64))],
        out_specs=full((2, 64)),
        out_shape=jax.ShapeDtypeStruct((2, 64), jnp.float32),
    )(xp, W1, b1r, g1r, bt1r, sums1, W2, b2r)

    o = pl.pallas_call(
        functools.partial(_final_pass, cnt=cnt),
        grid=(grid,),
        in_specs=[xspec, full((7, 16)), full((1, 16)), full((1, 16)),
                  full((1, 16)), full((2, T * 16)), full((32, 64)),
                  full((1, 64)), full((1, 64)), full((1, 64)), full((2, 64)),
                  full((128, FINAL)), full((1, FINAL))],
        out_specs=pl.BlockSpec((FINAL, Kt), lambda i: (0, i)),
        out_shape=jax.ShapeDtypeStruct((FINAL, N), jnp.float32),
    )(xp, W1, b1r, g1r, bt1r, sums1, W2, b2r, g2r, bt2r, sums2, Wf, bfr)

    out5 = pl.kernel(
        _sc_materialize,
        mesh=mesh,
        compiler_params=pltpu.CompilerParams(needs_layout_passes=False),
        out_type=jax.ShapeDtypeStruct((1, FINAL, D_, H_, W_), jnp.float32),
        scratch_types=[
            pltpu.VMEM((FPW * ROWP,), jnp.float32),
            pltpu.VMEM((56 * W_,), jnp.int32),
            pltpu.VMEM((56, W_), jnp.float32),
            pltpu.VMEM((56, W_), jnp.float32),
            pltpu.SemaphoreType.DMA,
        ],
    )(o.reshape(FINAL * N), idx)

    return out5


# speedup vs baseline: 1.0856x; 1.0050x over previous
import functools

import jax
import jax.numpy as jnp
from jax import lax
from jax.experimental import pallas as pl
from jax.experimental.pallas import tpu as pltpu
from jax.experimental.pallas import tpu_sc as plsc

D_, H_, W_ = 10, 200, 176
NVOX = D_ * H_ * W_
FINAL = 128
NEG = -1e30


Kt = 512
Tp = 40
T_REAL = 35


NPADS = 16384.0 * (Tp - T_REAL)


def _pad_t(xb):
    kt = xb.shape[0]
    return jnp.concatenate(
        [xb, jnp.zeros((kt, Tp - T_REAL, xb.shape[2]), xb.dtype)], axis=1)


def _layer1(xb, W1, b1):
    kt = xb.shape[0]
    x2 = xb.reshape(kt * Tp, 7)
    h = jnp.maximum(jnp.dot(x2, W1, preferred_element_type=jnp.float32) + b1, 0.0)
    return h


def _sums(h):
    s = jnp.sum(h, axis=0, keepdims=True)
    sq = jnp.sum(h * h, axis=0, keepdims=True)
    return jnp.concatenate([s, sq], axis=0)


def _stats_pass1(x_ref, W_ref, b_ref, out_ref):
    i = pl.program_id(0)
    h = jnp.maximum(
        jnp.dot(x_ref[...], W_ref[...], preferred_element_type=jnp.float32)
        + b_ref[...], 0.0)
    part = _sums(h)

    @pl.when(i == 0)
    def _():
        out_ref[...] = jnp.zeros_like(out_ref)

    out_ref[...] += part


def _bn_coeffs(sums, g, bt, b, cnt, pads):
    pc = jnp.maximum(b, 0.0)
    mean = (sums[0:1, :] - pads * pc) / cnt
    var = (sums[1:2, :] - pads * pc * pc) / cnt - mean * mean
    s = g * lax.rsqrt(var + 1e-5)
    t = bt - mean * s
    return s, t


def _negpad(row):
    t = lax.broadcasted_iota(jnp.int32, (1, Tp, 1), 1)
    return jnp.where(t < T_REAL, row.reshape(1, 1, -1), NEG)


def _vfe_split(h, kt, sums, g, bt, b, cnt, pads):
    c = h.shape[-1]
    s, t = _bn_coeffs(sums, g, bt, b, cnt, pads)
    hs = h * s
    hn = hs + t
    mp = jnp.max(hs.reshape(kt, Tp, c) + _negpad(t), axis=1)
    return hn, mp


def _pmask(xb):
    return (jnp.max(xb, axis=2, keepdims=True) != 0).astype(jnp.float32)


def _to_h2(xb, pm3, W1_ref, b1_ref, g1_ref, bt1_ref, s1_ref, W2_ref,
           b2_ref, cnt):
    kt = xb.shape[0]
    h1 = _layer1(xb, W1_ref[...], b1_ref[...])
    s1_16 = jnp.sum(s1_ref[...].reshape(2, T_REAL, 16), axis=1)
    hn1, mp1 = _vfe_split(h1, kt, s1_16, g1_ref[...], bt1_ref[...],
                          b1_ref[...], cnt, 0.0)
    W2 = W2_ref[...]
    a = jnp.dot(hn1, W2[0:16], preferred_element_type=jnp.float32)
    bmp = jnp.dot(mp1, W2[16:32], preferred_element_type=jnp.float32)
    pre = (a.reshape(kt, Tp, 64) + bmp[:, None, :]) * pm3
    h2 = jnp.maximum(pre + b2_ref[...].reshape(1, 1, 64), 0.0)
    return h2.reshape(kt * Tp, 64)


def _stats_pass2(x_ref, W1_ref, b1_ref, g1_ref, bt1_ref, s1_ref,
                 W2_ref, b2_ref, out_ref, *, cnt):
    i = pl.program_id(0)
    xb = _pad_t(x_ref[...])
    h2 = _to_h2(xb, _pmask(xb), W1_ref, b1_ref, g1_ref, bt1_ref, s1_ref,
                W2_ref, b2_ref, cnt)
    part = _sums(h2)

    @pl.when(i == 0)
    def _():
        out_ref[...] = jnp.zeros_like(out_ref)

    out_ref[...] += part


def _final_pass(x_ref, W1_ref, b1_ref, g1_ref, bt1_ref, s1_ref,
                W2_ref, b2_ref, g2_ref, bt2_ref, s2_ref,
                Wf_ref, bf_ref, o_ref, *, cnt):
    xb = _pad_t(x_ref[...])
    kt = xb.shape[0]
    pm3 = _pmask(xb)
    h2 = _to_h2(xb, pm3, W1_ref, b1_ref, g1_ref, bt1_ref, s1_ref,
                W2_ref, b2_ref, cnt)
    hn2, mp2 = _vfe_split(h2, kt, s2_ref[...], g2_ref[...], bt2_ref[...],
                          b2_ref[...], cnt, NPADS)
    Wf = Wf_ref[...]
    a2 = jnp.dot(hn2, Wf[0:64], preferred_element_type=jnp.float32)
    bmp2 = jnp.dot(mp2, Wf[64:128], preferred_element_type=jnp.float32)
    ym = (a2.reshape(kt, Tp, FINAL) + bmp2[:, None, :]) * pm3 \
        + _negpad(bf_ref[...])
    o_ref[...] = jnp.max(ym, axis=1).T


def _vid_kernel(c_ref, out_ref):
    c = c_ref[...]
    out_ref[...] = c[:, 0:1] * (H_ * W_) + c[:, 1:2] * W_ + c[:, 2:3]


NW = 32
VPW = NVOX // NW
VPW_PAD = 11008
NPTS = 16384
NGRP = NPTS // 16
SENT = NPTS
FPW = FINAL // NW
ROWP = 16400
VC = 8000
NVC = NVOX // VC


def _sc_build_idx(vid_hbm, idx_hbm, vid_v, table_v, keybuf_v, sem):
    wid = lax.axis_index("s") * 2 + lax.axis_index("c")
    base = wid * VPW
    pltpu.sync_copy(vid_hbm, vid_v)

    def init_body(j, _):
        table_v[pl.ds(j * 16, 16)] = jnp.full((16,), SENT, jnp.int32)
        return 0

    lax.fori_loop(0, VPW_PAD // 16, init_body, 0)
    keybuf_v[pl.ds(16, 16)] = jnp.full((16,), -1, jnp.int32)

    lane = lax.iota(jnp.int32, 16)

    def body(g, _):
        pi = g * 16 + lane
        vid = vid_v[pl.ds(g * 16, 16)]
        key = vid * 16 + lane
        sk, sv = plsc.sort_key_val(key, pi)
        keybuf_v[pl.ds(0, 16)] = sk
        nxt = keybuf_v[pl.ds(1, 16)]
        svid = lax.shift_right_arithmetic(sk, 4)
        nvid = lax.shift_right_arithmetic(nxt, 4)
        loc = svid - base
        m = (svid != nvid) & (loc >= 0) & (loc < VPW)
        locc = jnp.where(m, loc, 0)
        plsc.store_scatter(table_v, [locc], sv, mask=m)
        return 0

    lax.fori_loop(0, NGRP, body, 0)
    pltpu.sync_copy(table_v.at[pl.ds(0, VPW)], idx_hbm.at[pl.ds(base, VPW)])


def _sc_materialize(oT_hbm, idx_hbm, out_hbm,
                    orow_v, idxc_v, outc_v, outc1_v, sem):
    wid = lax.axis_index("s") * 2 + lax.axis_index("c")
    fb = wid * FPW
    for f in range(FPW):
        pltpu.sync_copy(oT_hbm.at[pl.ds((fb + f) * NPTS, NPTS)],
                        orow_v.at[pl.ds(f * ROWP, NPTS)])
        orow_v[pl.ds(f * ROWP + NPTS, 16)] = jnp.zeros((16,), jnp.float32)

    outc = [outc_v, outc1_v]
    gpr = W_ // 16

    def dloop(d, _):
        for h0, hn in ((0, 48), (48, 48), (96, 48), (144, H_ - 144)):
            voff = d * (H_ * W_) + h0 * W_
            pltpu.sync_copy(idx_hbm.at[pl.ds(voff, hn * W_)],
                            idxc_v.at[pl.ds(0, hn * W_)])
            ws = []
            for f in range(FPW):
                ob = outc[f % 2]
                if f >= 2:
                    ws[f - 2].wait()

                @plsc.parallel_loop(0, hn, 1, unroll=8)
                def rowloop(h):
                    for cc in range(gpr):
                        iv = idxc_v[pl.ds((h * gpr + cc) * 16, 16)]
                        vals = plsc.load_gather(orow_v, [iv + f * ROWP])
                        ob[h, pl.ds(cc * 16, 16)] = vals

                ws.append(pltpu.async_copy(
                    ob.at[pl.ds(0, hn)],
                    out_hbm.at[0, fb + f, d, pl.ds(h0, hn), :], sem))
            ws[-2].wait()
            ws[-1].wait()
        return 0

    lax.fori_loop(0, D_, dloop, 0)


def kernel(input, voxel_coor_buffer, W1, b1, g1, bt1, W2, b2, g2, bt2, Wf, bf):
    B, K, T, C = input.shape
    N = B * K
    cnt = float(N * T)

    xp = input.reshape(N, T, C)
    xpk = input.reshape(N, T * C)
    W1b = jnp.kron(jnp.eye(T, dtype=jnp.float32), W1)
    b1b = jnp.tile(b1.reshape(1, 16), (1, T))
    coor = voxel_coor_buffer.reshape(N, 3).astype(jnp.int32)

    b1r = b1.reshape(1, 16)
    g1r = g1.reshape(1, 16)
    bt1r = bt1.reshape(1, 16)
    b2r = b2.reshape(1, 64)
    g2r = g2.reshape(1, 64)
    bt2r = bt2.reshape(1, 64)
    bfr = bf.reshape(1, FINAL)

    grid = N // Kt
    full = lambda shp: pl.BlockSpec(shp, lambda i: (0,) * len(shp))
    xspec = pl.BlockSpec((Kt, T_REAL, C), lambda i: (i, 0, 0))

    vid = pl.pallas_call(
        _vid_kernel,
        in_specs=[pl.BlockSpec((N, 3), lambda: (0, 0))],
        out_specs=pl.BlockSpec((N, 1), lambda: (0, 0)),
        out_shape=jax.ShapeDtypeStruct((N, 1), jnp.int32),
    )(coor).reshape(N)

    mesh = plsc.VectorSubcoreMesh(core_axis_name="c", subcore_axis_name="s")

    idx = pl.kernel(
        _sc_build_idx,
        mesh=mesh,
        compiler_params=pltpu.CompilerParams(needs_layout_passes=False),
        out_type=jax.ShapeDtypeStruct((NVOX,), jnp.int32),
        scratch_types=[
            pltpu.VMEM((N,), jnp.int32),
            pltpu.VMEM((VPW_PAD,), jnp.int32),
            pltpu.VMEM((32,), jnp.int32),
            pltpu.SemaphoreType.DMA,
        ],
    )(vid)

    sums1 = pl.pallas_call(
        _stats_pass1,
        grid=(grid,),
        in_specs=[pl.BlockSpec((Kt, T * C), lambda i: (i, 0)),
                  full((T * C, T * 16)), full((1, T * 16))],
        out_specs=full((2, T * 16)),
        out_shape=jax.ShapeDtypeStruct((2, T * 16), jnp.float32),
    )(xpk, W1b, b1b)

    sums1, idx = lax.optimization_barrier((sums1, idx))

    sums2 = pl.pallas_call(
        functools.partial(_stats_pass2, cnt=cnt),
        grid=(grid,),
        in_specs=[xspec, full((7, 16)), full((1, 16)), full((1, 16)),
                  full((1, 16)), full((2, T * 16)), full((32, 64)),
                  full((1, 64))],
        out_specs=full((2, 64)),
        out_shape=jax.ShapeDtypeStruct((2, 64), jnp.float32),
    )(xp, W1, b1r, g1r, bt1r, sums1, W2, b2r)

    o = pl.pallas_call(
        functools.partial(_final_pass, cnt=cnt),
        grid=(grid,),
        in_specs=[xspec, full((7, 16)), full((1, 16)), full((1, 16)),
                  full((1, 16)), full((2, T * 16)), full((32, 64)),
                  full((1, 64)), full((1, 64)), full((1, 64)), full((2, 64)),
                  full((128, FINAL)), full((1, FINAL))],
        out_specs=pl.BlockSpec((FINAL, Kt), lambda i: (0, i)),
        out_shape=jax.ShapeDtypeStruct((FINAL, N), jnp.float32),
    )(xp, W1, b1r, g1r, bt1r, sums1, W2, b2r, g2r, bt2r, sums2, Wf, bfr)

    out5 = pl.kernel(
        _sc_materialize,
        mesh=mesh,
        compiler_params=pltpu.CompilerParams(needs_layout_passes=False),
        out_type=jax.ShapeDtypeStruct((1, FINAL, D_, H_, W_), jnp.float32),
        scratch_types=[
            pltpu.VMEM((FPW * ROWP,), jnp.float32),
            pltpu.VMEM((56 * W_,), jnp.int32),
            pltpu.VMEM((56, W_), jnp.float32),
            pltpu.VMEM((56, W_), jnp.float32),
            pltpu.SemaphoreType.DMA,
        ],
    )(o.reshape(FINAL * N), idx)

    return out5
